# baseline (speedup 1.0000x reference)
import functools

import jax
import jax.numpy as jnp
from jax import lax
from jax.experimental import pallas as pl
from jax.experimental.pallas import tpu as pltpu
from jax.experimental.pallas import tpu_sc as plsc

N = 10000
G = 64
NC = 2
NS = 16
NW = NC * NS
CHUNK = 128
N_PAD = 10240
DUMP = N

_F32 = jnp.float32


def _dot(a, b):
    return lax.dot_general(a, b, (((1,), (0,)), ((), ())),
                           precision=lax.Precision.DEFAULT,
                           preferred_element_type=_F32)


def _sc_scatter_add(h, src2d, dst2d, d, cpt0, cpt1):
    rows_per_tile = N_PAD // NS
    zrows = CHUNK
    pch = 32
    passes = -(-max(cpt0, cpt1) // pch)

    mesh = plsc.VectorSubcoreMesh(core_axis_name="c", subcore_axis_name="s")

    @functools.partial(
        pl.kernel,
        out_type=jax.ShapeDtypeStruct((NC, N_PAD, d), _F32),
        mesh=mesh,
        scratch_types=[
            pltpu.VMEM((pch, CHUNK), jnp.int32),
            pltpu.VMEM((pch, CHUNK), jnp.int32),
            pltpu.VMEM((2, CHUNK, d), _F32),
            pltpu.VMEM_SHARED((N_PAD, d), _F32),
            pltpu.SemaphoreType.DMA,
        ],
    )
    def k(h_hbm, src_hbm, dst_hbm, out_hbm, sidx, didx, rows, agg, sem):
        c = lax.axis_index("c")
        s = lax.axis_index("s")

        zv = jnp.zeros((16,), _F32)
        zbuf = rows.at[0]

        def zero_row(r, _):
            for j in range(d // 16):
                zbuf[r, pl.ds(16 * j, 16)] = zv
            return 0

        lax.fori_loop(0, zrows, zero_row, 0)
        for t in range(rows_per_tile // zrows):
            pltpu.sync_copy(zbuf, agg.at[pl.ds(s * rows_per_tile + t * zrows, zrows)])
        plsc.subcore_barrier()

        my_cpt = jnp.where(c == 0, cpt0, cpt1)
        tile_base = jnp.where(c == 0, s * cpt0, 16 * cpt0 + s * cpt1)

        for p in range(passes):
            @pl.when(p * pch < my_cpt)
            def _():
                base = pl.multiple_of(tile_base + p * pch, 8)
                pltpu.sync_copy(src_hbm.at[pl.ds(base, pch)], sidx)
                pltpu.sync_copy(dst_hbm.at[pl.ds(base, pch)], didx)

                pltpu.async_copy(h_hbm.at[sidx.at[0]], rows.at[0], sem)

                def body(j, _):
                    nxt = j + 1

                    @pl.when(nxt < pch)
                    def _():
                        pltpu.async_copy(h_hbm.at[sidx.at[nxt]],
                                         rows.at[nxt % 2], sem)

                    buf = rows.at[j % 2]
                    pltpu.make_async_copy(h_hbm.at[sidx.at[j]], buf,
                                          sem).wait()
                    pltpu.sync_copy(buf, agg.at[didx.at[j]], add=True)
                    return 0

                lax.fori_loop(0, pch, body, 0)
        plsc.subcore_barrier()

        for t in range(rows_per_tile // zrows):
            r0 = s * rows_per_tile + t * zrows
            pltpu.sync_copy(agg.at[pl.ds(r0, zrows)], zbuf)
            pltpu.sync_copy(zbuf, out_hbm.at[c].at[pl.ds(r0, zrows)])

    return k(h, src2d, dst2d)


def _tc_layer_body(h_ref, agg_ref, eps_ref, w1_ref, b1_ref, g1_ref, be1_ref,
                   w2_ref, b2_ref, gbn_ref, bbn_ref, batch_ref,
                   hout_ref, pool_ref):
    din = w1_ref.shape[0]
    h = h_ref[:, :din]
    z = (1.0 + eps_ref[0, 0]) * h + (agg_ref[0, :N, :din]
                                     + agg_ref[1, :N, :din])
    t = _dot(z, w1_ref[:]) + b1_ref[:]
    m = jnp.mean(t, axis=0, keepdims=True)
    v = jnp.mean((t - m) ** 2, axis=0, keepdims=True)
    t = (t - m) * lax.rsqrt(v + 1e-5) * g1_ref[:] + be1_ref[:]
    t = jnp.maximum(t, 0.0)
    u = _dot(t, w2_ref[:]) + b2_ref[:]
    m2 = jnp.mean(u, axis=0, keepdims=True)
    v2 = jnp.mean((u - m2) ** 2, axis=0, keepdims=True)
    h2 = jnp.maximum((u - m2) * lax.rsqrt(v2 + 1e-5) * gbn_ref[:] + bbn_ref[:],
                     0.0)
    hout_ref[:] = jnp.concatenate(
        [h2, jnp.zeros((N, hout_ref.shape[1] - h2.shape[1]), _F32)], axis=1)
    gid = lax.broadcasted_iota(jnp.int32, (G, N), 0)
    onehot = (gid == batch_ref[:]).astype(_F32)
    pool_ref[:] = _dot(onehot, h2)


def _tc_layer(h, agg, p, batch2d, hdim):
    return pl.pallas_call(
        _tc_layer_body,
        out_shape=[
            jax.ShapeDtypeStruct((N, 128), _F32),
            jax.ShapeDtypeStruct((G, hdim), _F32),
        ],
    )(h, agg,
      p['eps'].reshape(1, 1), p['W1'], p['b1'].reshape(1, -1),
      p['g1'].reshape(1, -1), p['be1'].reshape(1, -1),
      p['W2'], p['b2'].reshape(1, -1), p['gbn'].reshape(1, -1),
      p['bbn'].reshape(1, -1), batch2d)


def _tc_head_body(p1_ref, p2_ref, p3_ref, wc1_ref, bc1_ref, wc2_ref, bc2_ref,
                  out_ref):
    hdim = p1_ref.shape[1]
    hc = (_dot(p1_ref[:], wc1_ref[0 * hdim:1 * hdim, :])
          + _dot(p2_ref[:], wc1_ref[1 * hdim:2 * hdim, :])
          + _dot(p3_ref[:], wc1_ref[2 * hdim:3 * hdim, :])
          + bc1_ref[:])
    hc = jnp.maximum(hc, 0.0)
    out_ref[:] = _dot(hc, wc2_ref[:]) + bc2_ref[:]


def _tc_head(pools, wc1, bc1, wc2, bc2):
    wc2p = jnp.zeros((wc2.shape[0], 128), _F32).at[:, :wc2.shape[1]].set(wc2)
    bc2p = jnp.zeros((1, 128), _F32).at[0, :wc2.shape[1]].set(bc2)
    out = pl.pallas_call(
        _tc_head_body,
        out_shape=jax.ShapeDtypeStruct((G, 128), _F32),
    )(pools[0], pools[1], pools[2], wc1, bc1.reshape(1, -1), wc2p, bc2p)
    return out[:, :wc2.shape[1]]


def kernel(x, edge_index, batch, params):
    E = edge_index.shape[1]
    tot_pt = -(-E // (NS * CHUNK * 32)) * 32
    cpt0 = (tot_pt * 3 // 5) // 32 * 32
    cpt1 = tot_pt - cpt0
    e_pad = NS * CHUNK * tot_pt
    src = jnp.concatenate([edge_index[0],
                           jnp.zeros((e_pad - E,), jnp.int32)])
    dst = jnp.concatenate([edge_index[1],
                           DUMP + (jnp.arange(e_pad - E, dtype=jnp.int32)
                                   % (N_PAD - N))])
    src2d = src.reshape(e_pad // CHUNK, CHUNK)
    dst2d = dst.reshape(e_pad // CHUNK, CHUNK)
    batch2d = batch.reshape(1, N)

    h = x
    pools = []
    for p in params['layers']:
        agg = _sc_scatter_add(h, src2d, dst2d, 128, cpt0, cpt1)
        h, pool = _tc_layer(h, agg, p, batch2d, p['W2'].shape[1])
        pools.append(pool)
    return _tc_head(pools, params['Wc1'], params['bc1'],
                    params['Wc2'], params['bc2'])


# speedup vs baseline: 1.0423x; 1.0423x over previous
import functools

import jax
import jax.numpy as jnp
from jax import lax
from jax.experimental import pallas as pl
from jax.experimental.pallas import tpu as pltpu
from jax.experimental.pallas import tpu_sc as plsc

N = 10000
G = 64
NC = 2
NS = 16
NW = NC * NS
CHUNK = 128
N_PAD = 10240
DUMP = N

_F32 = jnp.float32


def _dot(a, b):
    return lax.dot_general(a, b, (((1,), (0,)), ((), ())),
                           precision=lax.Precision.DEFAULT,
                           preferred_element_type=_F32)


def _sc_scatter_add(h, src2d, dst2d, d, cpt0, cpt1):
    rows_per_tile = N_PAD // NS
    zrows = CHUNK
    pch = 32
    passes = -(-max(cpt0, cpt1) // pch)

    mesh = plsc.VectorSubcoreMesh(core_axis_name="c", subcore_axis_name="s")

    @functools.partial(
        pl.kernel,
        out_type=jax.ShapeDtypeStruct((NC, N_PAD, d), _F32),
        mesh=mesh,
        scratch_types=[
            pltpu.VMEM((pch, CHUNK), jnp.int32),
            pltpu.VMEM((pch, CHUNK), jnp.int32),
            pltpu.VMEM((2, CHUNK, d), _F32),
            pltpu.VMEM_SHARED((N_PAD, d), _F32),
            pltpu.SemaphoreType.DMA,
        ],
    )
    def k(h_hbm, src_hbm, dst_hbm, out_hbm, sidx, didx, rows, agg, sem):
        c = lax.axis_index("c")
        s = lax.axis_index("s")

        zv = jnp.zeros((16,), _F32)
        zbuf = rows.at[0]

        def zero_row(r, _):
            for j in range(d // 16):
                zbuf[r, pl.ds(16 * j, 16)] = zv
            return 0

        lax.fori_loop(0, zrows, zero_row, 0)
        for t in range(rows_per_tile // zrows):
            pltpu.sync_copy(zbuf, agg.at[pl.ds(s * rows_per_tile + t * zrows, zrows)])
        plsc.subcore_barrier()

        my_cpt = jnp.where(c == 0, cpt0, cpt1)
        tile_base = jnp.where(c == 0, s * cpt0, 16 * cpt0 + s * cpt1)

        for p in range(passes):
            @pl.when(p * pch < my_cpt)
            def _():
                base = pl.multiple_of(tile_base + p * pch, 8)
                pltpu.sync_copy(src_hbm.at[pl.ds(base, pch)], sidx)
                pltpu.sync_copy(dst_hbm.at[pl.ds(base, pch)], didx)

                pltpu.async_copy(h_hbm.at[sidx.at[0]], rows.at[0], sem)

                def body(j, _):
                    nxt = j + 1

                    @pl.when(nxt < pch)
                    def _():
                        pltpu.async_copy(h_hbm.at[sidx.at[nxt]],
                                         rows.at[nxt % 2], sem)

                    buf = rows.at[j % 2]
                    pltpu.make_async_copy(h_hbm.at[sidx.at[j]], buf,
                                          sem).wait()
                    pltpu.sync_copy(buf, agg.at[didx.at[j]], add=True)
                    return 0

                lax.fori_loop(0, pch, body, 0)
        plsc.subcore_barrier()

        for t in range(rows_per_tile // zrows):
            r0 = s * rows_per_tile + t * zrows
            pltpu.sync_copy(agg.at[pl.ds(r0, zrows)], zbuf)
            pltpu.sync_copy(zbuf, out_hbm.at[c].at[pl.ds(r0, zrows)])

    return k(h, src2d, dst2d)


def _tc_layer_body(h_ref, agg_ref, eps_ref, w1_ref, b1_ref, g1_ref, be1_ref,
                   w2_ref, b2_ref, gbn_ref, bbn_ref, batch_ref,
                   hout_ref, pool_ref):
    din = w1_ref.shape[0]
    h = h_ref[:, :din]
    z = (1.0 + eps_ref[0, 0]) * h + (agg_ref[0, :N, :din]
                                     + agg_ref[1, :N, :din])
    t = _dot(z, w1_ref[:]) + b1_ref[:]
    m = jnp.mean(t, axis=0, keepdims=True)
    v = jnp.mean((t - m) ** 2, axis=0, keepdims=True)
    t = (t - m) * lax.rsqrt(v + 1e-5) * g1_ref[:] + be1_ref[:]
    t = jnp.maximum(t, 0.0)
    u = _dot(t, w2_ref[:]) + b2_ref[:]
    m2 = jnp.mean(u, axis=0, keepdims=True)
    v2 = jnp.mean((u - m2) ** 2, axis=0, keepdims=True)
    h2 = jnp.maximum((u - m2) * lax.rsqrt(v2 + 1e-5) * gbn_ref[:] + bbn_ref[:],
                     0.0)
    hout_ref[:] = jnp.concatenate(
        [h2, jnp.zeros((N, hout_ref.shape[1] - h2.shape[1]), _F32)], axis=1)
    gid = lax.broadcasted_iota(jnp.int32, (G, N), 0)
    onehot = (gid == batch_ref[:]).astype(_F32)
    pool_ref[:] = _dot(onehot, h2)


def _tc_layer(h, agg, p, batch2d, hdim):
    return pl.pallas_call(
        _tc_layer_body,
        out_shape=[
            jax.ShapeDtypeStruct((N, 128), _F32),
            jax.ShapeDtypeStruct((G, hdim), _F32),
        ],
    )(h, agg,
      p['eps'].reshape(1, 1), p['W1'], p['b1'].reshape(1, -1),
      p['g1'].reshape(1, -1), p['be1'].reshape(1, -1),
      p['W2'], p['b2'].reshape(1, -1), p['gbn'].reshape(1, -1),
      p['bbn'].reshape(1, -1), batch2d)


def _tc_head_body(p1_ref, p2_ref, p3_ref, wc1_ref, bc1_ref, wc2_ref, bc2_ref,
                  out_ref):
    hdim = p1_ref.shape[1]
    hc = (_dot(p1_ref[:], wc1_ref[0 * hdim:1 * hdim, :])
          + _dot(p2_ref[:], wc1_ref[1 * hdim:2 * hdim, :])
          + _dot(p3_ref[:], wc1_ref[2 * hdim:3 * hdim, :])
          + bc1_ref[:])
    hc = jnp.maximum(hc, 0.0)
    out_ref[:] = _dot(hc, wc2_ref[:]) + bc2_ref[:]


def _tc_head(pools, wc1, bc1, wc2, bc2):
    wc2p = jnp.zeros((wc2.shape[0], 128), _F32).at[:, :wc2.shape[1]].set(wc2)
    bc2p = jnp.zeros((1, 128), _F32).at[0, :wc2.shape[1]].set(bc2)
    out = pl.pallas_call(
        _tc_head_body,
        out_shape=jax.ShapeDtypeStruct((G, 128), _F32),
    )(pools[0], pools[1], pools[2], wc1, bc1.reshape(1, -1), wc2p, bc2p)
    return out[:, :wc2.shape[1]]


def kernel(x, edge_index, batch, params):
    E = edge_index.shape[1]
    tot_pt = -(-E // (NS * CHUNK * 32)) * 32
    cpt0 = (tot_pt * 4 // 5) // 32 * 32
    cpt1 = tot_pt - cpt0
    e_pad = NS * CHUNK * tot_pt
    src = jnp.concatenate([edge_index[0],
                           jnp.zeros((e_pad - E,), jnp.int32)])
    dst = jnp.concatenate([edge_index[1],
                           DUMP + (jnp.arange(e_pad - E, dtype=jnp.int32)
                                   % (N_PAD - N))])
    src2d = src.reshape(e_pad // CHUNK, CHUNK)
    dst2d = dst.reshape(e_pad // CHUNK, CHUNK)
    batch2d = batch.reshape(1, N)

    h = x
    pools = []
    for p in params['layers']:
        agg = _sc_scatter_add(h, src2d, dst2d, 128, cpt0, cpt1)
        h, pool = _tc_layer(h, agg, p, batch2d, p['W2'].shape[1])
        pools.append(pool)
    return _tc_head(pools, params['Wc1'], params['bc1'],
                    params['Wc2'], params['bc2'])
